# baseline (speedup 1.0000x reference)
import functools

import jax
import jax.numpy as jnp
from jax import lax
from jax.experimental import pallas as pl
from jax.experimental.pallas import tpu as pltpu
from jax.experimental.pallas import tpu_sc as plsc

_HG = 64
_L = 4
_G = 2
_NC = 2
_NS = 16
_NW = _NC * _NS
_CR = 128
_CM = 3
_CH = _CR * _CM
_NBUF = 2
_EPS = 1e-5


def _make_edge_pass(n, k_chunks):
    mesh = plsc.VectorSubcoreMesh(core_axis_name="c", subcore_axis_name="s")
    rps = (n // _NS) // 8 * 8
    tail = n - rps * _NS

    @functools.partial(
        pl.kernel,
        out_type=jax.ShapeDtypeStruct((_NC, n, _HG), jnp.float32),
        mesh=mesh,
        scratch_types=[
            pltpu.VMEM((k_chunks, _CH), jnp.int32),
            pltpu.VMEM((k_chunks, _CH), jnp.int32),
            pltpu.VMEM((_NBUF, _CH, 16), jnp.float32),
            pltpu.VMEM((_NBUF, _CH, _HG), jnp.float32),
            pltpu.VMEM_SHARED((n, _HG), jnp.float32),
            pltpu.SemaphoreType.DMA,
            pltpu.SemaphoreType.DMA((_NBUF,)),
            pltpu.SemaphoreType.DMA((_NBUF,)),
            pltpu.SemaphoreType.DMA((_NBUF,)),
        ],
        compiler_params=pltpu.CompilerParams(use_tc_tiling_on_sc=False),
    )
    def edge_pass(m_hbm, src_hbm, dst_hbm, w16_hbm, zero_hbm,
                  out_hbm, src_v, dst_v, w16_v, rows_v, agg_sh,
                  sem_in, sem_g, sem_s, sem_w):
        c = lax.axis_index("c")
        s = lax.axis_index("s")
        wid = c * _NS + s
        cp1 = pltpu.async_copy(src_hbm.at[wid], src_v, sem_in)
        cp2 = pltpu.async_copy(dst_hbm.at[wid], dst_v, sem_in)
        r0 = s * rps
        pltpu.sync_copy(zero_hbm.at[pl.ds(r0, rps)],
                        agg_sh.at[pl.ds(r0, rps)])
        if tail:
            @pl.when(s == _NS - 1)
            def _():
                pltpu.sync_copy(zero_hbm.at[pl.ds(rps * _NS, tail)],
                                agg_sh.at[pl.ds(rps * _NS, tail)])

        cp1.wait()
        cp2.wait()
        plsc.subcore_barrier()

        def block_body(q, carry):
            @pl.when(q > 0)
            def _():
                for u in range(_NBUF):
                    pltpu.make_async_copy(
                        rows_v.at[u], agg_sh.at[dst_v.at[0]], sem_s.at[u]
                    ).wait()

            for u in range(_NBUF):
                k = q * _NBUF + u
                pltpu.async_copy(m_hbm.at[src_v.at[k]],
                                 rows_v.at[u], sem_g.at[u])
                pltpu.async_copy(w16_hbm.at[wid, k], w16_v.at[u],
                                 sem_w.at[u])
            for u in range(_NBUF):
                k = q * _NBUF + u
                pltpu.make_async_copy(m_hbm.at[src_v.at[k]],
                                      rows_v.at[u], sem_g.at[u]).wait()
                pltpu.make_async_copy(w16_hbm.at[wid, k], w16_v.at[u],
                                      sem_w.at[u]).wait()

                def scale_body(i, carry2):
                    for t in range(2):
                        r = i * 2 + t
                        wv = w16_v[u, r]
                        for j in range(_HG // 16):
                            sl = pl.ds(j * 16, 16)
                            rows_v[u, r, sl] = rows_v[u, r, sl] * wv
                    return carry2

                lax.fori_loop(0, _CH // 2, scale_body, 0)
                pltpu.async_copy(rows_v.at[u],
                                 agg_sh.at[dst_v.at[k]],
                                 sem_s.at[u], add=True)
            return carry

        lax.fori_loop(0, k_chunks // _NBUF, block_body, 0)
        for u in range(_NBUF):
            pltpu.make_async_copy(
                rows_v.at[u], agg_sh.at[dst_v.at[0]], sem_s.at[u]
            ).wait()
        plsc.subcore_barrier()
        pltpu.sync_copy(agg_sh.at[pl.ds(r0, rps)],
                        out_hbm.at[c, pl.ds(r0, rps)])
        if tail:
            @pl.when(s == _NS - 1)
            def _():
                pltpu.sync_copy(agg_sh.at[pl.ds(rps * _NS, tail)],
                                out_hbm.at[c, pl.ds(rps * _NS, tail)])

    return edge_pass


def _ln_relu(h, g, b):
    mu = jnp.mean(h, axis=-1, keepdims=True)
    d = h - mu
    var = jnp.mean(d * d, axis=-1, keepdims=True)
    return jnp.maximum(d * lax.rsqrt(var + _EPS) * g + b, 0.0)


def _pre_body(x_ref, w1_ref, b1_ref, lng_ref, lnb_ref, wc_ref,
              y0_ref, y1_ref, m_ref):
    h = jnp.dot(x_ref[...], w1_ref[...],
                preferred_element_type=jnp.float32) + b1_ref[...]
    y0_ref[...] = h[:, :_HG]
    y1 = h[:, _HG:]
    y1_ref[...] = y1
    z = _ln_relu(y1, lng_ref[...], lnb_ref[...])
    m_ref[...] = jnp.dot(z, wc_ref[...], preferred_element_type=jnp.float32)


def _step_body(agg_ref, base_ref, cb_ref, lng_ref, lnb_ref, wc_ref,
               y_ref, m_ref):
    y = base_ref[...] + agg_ref[0] + agg_ref[1] + cb_ref[...]
    y_ref[...] = y
    z = _ln_relu(y, lng_ref[...], lnb_ref[...])
    m_ref[...] = jnp.dot(z, wc_ref[...], preferred_element_type=jnp.float32)


def _last_body(agg_ref, base_ref, cb_ref, y0_ref_in, fng_ref, fnb_ref,
               w2_ref, b2_ref, out_ref):
    y1 = base_ref[...] + agg_ref[0] + agg_ref[1] + cb_ref[...]
    h = jnp.concatenate([y0_ref_in[...], y1], axis=-1)
    z = _ln_relu(h, fng_ref[...], fnb_ref[...])
    out_ref[...] = jnp.dot(z, w2_ref[...],
                           preferred_element_type=jnp.float32) + b2_ref[...]


def kernel(x, edge_index_graph, edge_weight_graph, W1, b1, ln_g, ln_b,
           convW, convB, fn_g, fn_b, W2, b2):
    n = x.shape[0]
    e = edge_weight_graph.shape[0]
    out_dim = W2.shape[1]
    f32 = jnp.float32

    k_chunks = -(-e // (_NW * _CH))
    k_chunks = -(-k_chunks // _NBUF) * _NBUF
    ep = _NW * k_chunks * _CH
    spread = (jnp.arange(ep - e, dtype=jnp.int32) * 64) % n
    src = jnp.concatenate([edge_index_graph[0], spread]).reshape(
        _NW, k_chunks, _CH)
    dst = jnp.concatenate([edge_index_graph[1], spread]).reshape(
        _NW, k_chunks, _CH)
    wgt = jnp.pad(edge_weight_graph, (0, ep - e))
    w16 = jnp.broadcast_to(wgt[:, None], (ep, 16)).reshape(
        _NW, k_chunks, _CH, 16)
    zeros = jnp.zeros((n, _HG), f32)

    edge_pass = _make_edge_pass(n, k_chunks)
    sds = jax.ShapeDtypeStruct

    y0, y1, m = pl.pallas_call(
        _pre_body,
        out_shape=(sds((n, _HG), f32), sds((n, _HG), f32), sds((n, _HG), f32)),
    )(x, W1, b1[None], ln_g[0, 0][None], ln_b[0, 0][None], convW[0, 0])

    steps = [(l, g) for l in range(_L) for g in range(_G)]
    for idx, (l, g) in enumerate(steps):
        base = y0 if g == 0 else y1
        agg2 = edge_pass(m, src, dst, w16, zeros)
        if idx + 1 < len(steps):
            ln_, gn_ = steps[idx + 1]
            y, m = pl.pallas_call(
                _step_body,
                out_shape=(sds((n, _HG), f32), sds((n, _HG), f32)),
            )(agg2, base, convB[l, g][None], ln_g[ln_, gn_][None],
              ln_b[ln_, gn_][None], convW[ln_, gn_])
            if g == 0:
                y0 = y
            else:
                y1 = y
        else:
            out = pl.pallas_call(
                _last_body,
                out_shape=sds((n, out_dim), f32),
            )(agg2, base, convB[l, g][None], y0, fn_g[None], fn_b[None],
              W2, b2[None])
    return out


# speedup vs baseline: 1.0485x; 1.0485x over previous
import functools

import jax
import jax.numpy as jnp
from jax import lax
from jax.experimental import pallas as pl
from jax.experimental.pallas import tpu as pltpu
from jax.experimental.pallas import tpu_sc as plsc

_HG = 64
_L = 4
_G = 2
_NC = 2
_NS = 16
_NW = _NC * _NS
_CH = 256
_NBUF = 3
_EPS = 1e-5


def _make_edge_pass(n, k_chunks):
    mesh = plsc.VectorSubcoreMesh(core_axis_name="c", subcore_axis_name="s")
    rps = (n // _NS) // 8 * 8
    tail = n - rps * _NS

    @functools.partial(
        pl.kernel,
        out_type=jax.ShapeDtypeStruct((_NC, n, _HG), jnp.float32),
        mesh=mesh,
        scratch_types=[
            pltpu.VMEM((k_chunks, _CH), jnp.int32),
            pltpu.VMEM((k_chunks, _CH), jnp.int32),
            pltpu.VMEM((_NBUF, _CH, 16), jnp.float32),
            pltpu.VMEM((_NBUF, _CH, _HG), jnp.float32),
            pltpu.VMEM_SHARED((n, _HG), jnp.float32),
            pltpu.SemaphoreType.DMA,
            pltpu.SemaphoreType.DMA((_NBUF,)),
            pltpu.SemaphoreType.DMA((_NBUF,)),
            pltpu.SemaphoreType.DMA((_NBUF,)),
        ],
        compiler_params=pltpu.CompilerParams(use_tc_tiling_on_sc=False),
    )
    def edge_pass(m_hbm, src_hbm, dst_hbm, w16_hbm, zero_hbm,
                  out_hbm, src_v, dst_v, w16_v, rows_v, agg_sh,
                  sem_in, sem_g, sem_s, sem_w):
        c = lax.axis_index("c")
        s = lax.axis_index("s")
        wid = c * _NS + s
        cp1 = pltpu.async_copy(src_hbm.at[wid], src_v, sem_in)
        cp2 = pltpu.async_copy(dst_hbm.at[wid], dst_v, sem_in)
        r0 = s * rps
        pltpu.sync_copy(zero_hbm.at[pl.ds(r0, rps)],
                        agg_sh.at[pl.ds(r0, rps)])
        if tail:
            @pl.when(s == _NS - 1)
            def _():
                pltpu.sync_copy(zero_hbm.at[pl.ds(rps * _NS, tail)],
                                agg_sh.at[pl.ds(rps * _NS, tail)])

        cp1.wait()
        cp2.wait()
        plsc.subcore_barrier()

        def block_body(q, carry):
            @pl.when(q > 0)
            def _():
                for u in range(_NBUF):
                    pltpu.make_async_copy(
                        rows_v.at[u], agg_sh.at[dst_v.at[0]], sem_s.at[u]
                    ).wait()

            for u in range(_NBUF):
                k = q * _NBUF + u
                pltpu.async_copy(m_hbm.at[src_v.at[k]],
                                 rows_v.at[u], sem_g.at[u])
                pltpu.async_copy(w16_hbm.at[wid, k], w16_v.at[u],
                                 sem_w.at[u])
            for u in range(_NBUF):
                k = q * _NBUF + u
                pltpu.make_async_copy(m_hbm.at[src_v.at[k]],
                                      rows_v.at[u], sem_g.at[u]).wait()
                pltpu.make_async_copy(w16_hbm.at[wid, k], w16_v.at[u],
                                      sem_w.at[u]).wait()

                def scale_body(i, carry2):
                    for t in range(2):
                        r = i * 2 + t
                        wv = w16_v[u, r]
                        for j in range(_HG // 16):
                            sl = pl.ds(j * 16, 16)
                            rows_v[u, r, sl] = rows_v[u, r, sl] * wv
                    return carry2

                lax.fori_loop(0, _CH // 2, scale_body, 0)
                pltpu.async_copy(rows_v.at[u],
                                 agg_sh.at[dst_v.at[k]],
                                 sem_s.at[u], add=True)
            return carry

        lax.fori_loop(0, k_chunks // _NBUF, block_body, 0)
        for u in range(_NBUF):
            pltpu.make_async_copy(
                rows_v.at[u], agg_sh.at[dst_v.at[0]], sem_s.at[u]
            ).wait()
        plsc.subcore_barrier()
        pltpu.sync_copy(agg_sh.at[pl.ds(r0, rps)],
                        out_hbm.at[c, pl.ds(r0, rps)])
        if tail:
            @pl.when(s == _NS - 1)
            def _():
                pltpu.sync_copy(agg_sh.at[pl.ds(rps * _NS, tail)],
                                out_hbm.at[c, pl.ds(rps * _NS, tail)])

    return edge_pass


def _ln_relu(h, g, b):
    mu = jnp.mean(h, axis=-1, keepdims=True)
    d = h - mu
    var = jnp.mean(d * d, axis=-1, keepdims=True)
    return jnp.maximum(d * lax.rsqrt(var + _EPS) * g + b, 0.0)


def _pre_body(x_ref, w1_ref, b1_ref, lng_ref, lnb_ref, wc_ref,
              y0_ref, y1_ref, m_ref):
    h = jnp.dot(x_ref[...], w1_ref[...],
                preferred_element_type=jnp.float32) + b1_ref[...]
    y0_ref[...] = h[:, :_HG]
    y1 = h[:, _HG:]
    y1_ref[...] = y1
    z = _ln_relu(y1, lng_ref[...], lnb_ref[...])
    m_ref[...] = jnp.dot(z, wc_ref[...], preferred_element_type=jnp.float32)


def _step_body(agg_ref, base_ref, cb_ref, lng_ref, lnb_ref, wc_ref,
               y_ref, m_ref):
    y = base_ref[...] + agg_ref[0] + agg_ref[1] + cb_ref[...]
    y_ref[...] = y
    z = _ln_relu(y, lng_ref[...], lnb_ref[...])
    m_ref[...] = jnp.dot(z, wc_ref[...], preferred_element_type=jnp.float32)


def _last_body(agg_ref, base_ref, cb_ref, y0_ref_in, fng_ref, fnb_ref,
               w2_ref, b2_ref, out_ref):
    y1 = base_ref[...] + agg_ref[0] + agg_ref[1] + cb_ref[...]
    h = jnp.concatenate([y0_ref_in[...], y1], axis=-1)
    z = _ln_relu(h, fng_ref[...], fnb_ref[...])
    out_ref[...] = jnp.dot(z, w2_ref[...],
                           preferred_element_type=jnp.float32) + b2_ref[...]


def kernel(x, edge_index_graph, edge_weight_graph, W1, b1, ln_g, ln_b,
           convW, convB, fn_g, fn_b, W2, b2):
    n = x.shape[0]
    e = edge_weight_graph.shape[0]
    out_dim = W2.shape[1]
    f32 = jnp.float32

    k_chunks = -(-e // (_NW * _CH))
    k_chunks = -(-k_chunks // _NBUF) * _NBUF
    ep = _NW * k_chunks * _CH
    spread = (jnp.arange(ep - e, dtype=jnp.int32) * 64) % n
    src = jnp.concatenate([edge_index_graph[0], spread]).reshape(
        _NW, k_chunks, _CH)
    dst = jnp.concatenate([edge_index_graph[1], spread]).reshape(
        _NW, k_chunks, _CH)
    wgt = jnp.pad(edge_weight_graph, (0, ep - e))
    w16 = jnp.broadcast_to(wgt[:, None], (ep, 16)).reshape(
        _NW, k_chunks, _CH, 16)
    zeros = jnp.zeros((n, _HG), f32)

    edge_pass = _make_edge_pass(n, k_chunks)
    sds = jax.ShapeDtypeStruct

    y0, y1, m = pl.pallas_call(
        _pre_body,
        out_shape=(sds((n, _HG), f32), sds((n, _HG), f32), sds((n, _HG), f32)),
    )(x, W1, b1[None], ln_g[0, 0][None], ln_b[0, 0][None], convW[0, 0])

    steps = [(l, g) for l in range(_L) for g in range(_G)]
    for idx, (l, g) in enumerate(steps):
        base = y0 if g == 0 else y1
        agg2 = edge_pass(m, src, dst, w16, zeros)
        if idx + 1 < len(steps):
            ln_, gn_ = steps[idx + 1]
            y, m = pl.pallas_call(
                _step_body,
                out_shape=(sds((n, _HG), f32), sds((n, _HG), f32)),
            )(agg2, base, convB[l, g][None], ln_g[ln_, gn_][None],
              ln_b[ln_, gn_][None], convW[ln_, gn_])
            if g == 0:
                y0 = y
            else:
                y1 = y
        else:
            out = pl.pallas_call(
                _last_body,
                out_shape=sds((n, out_dim), f32),
            )(agg2, base, convB[l, g][None], y0, fn_g[None], fn_b[None],
              W2, b2[None])
    return out


# speedup vs baseline: 1.1320x; 1.0796x over previous
import functools

import jax
import jax.numpy as jnp
from jax import lax
from jax.experimental import pallas as pl
from jax.experimental.pallas import tpu as pltpu
from jax.experimental.pallas import tpu_sc as plsc

_HG = 64
_L = 4
_G = 2
_NC = 2
_NS = 16
_NW = _NC * _NS
_CH = 128
_NBUF = 6
_EPS = 1e-5


def _make_edge_pass(n, k_chunks):
    mesh = plsc.VectorSubcoreMesh(core_axis_name="c", subcore_axis_name="s")
    rps = (n // _NS) // 8 * 8
    tail = n - rps * _NS

    @functools.partial(
        pl.kernel,
        out_type=jax.ShapeDtypeStruct((_NC, n, _HG), jnp.float32),
        mesh=mesh,
        scratch_types=[
            pltpu.VMEM((k_chunks, _CH), jnp.int32),
            pltpu.VMEM((k_chunks, _CH), jnp.int32),
            pltpu.VMEM((_NBUF, _CH, 16), jnp.float32),
            pltpu.VMEM((_NBUF, _CH, _HG), jnp.float32),
            pltpu.VMEM_SHARED((n, _HG), jnp.float32),
            pltpu.SemaphoreType.DMA,
            pltpu.SemaphoreType.DMA((_NBUF,)),
            pltpu.SemaphoreType.DMA((_NBUF,)),
            pltpu.SemaphoreType.DMA((_NBUF,)),
        ],
        compiler_params=pltpu.CompilerParams(use_tc_tiling_on_sc=False),
    )
    def edge_pass(m_hbm, src_hbm, dst_hbm, w16_hbm, base_hbm, zero_hbm,
                  out_hbm, src_v, dst_v, w16_v, rows_v, agg_sh,
                  sem_in, sem_g, sem_s, sem_w):
        c = lax.axis_index("c")
        s = lax.axis_index("s")
        wid = c * _NS + s
        cp1 = pltpu.async_copy(src_hbm.at[wid], src_v, sem_in)
        cp2 = pltpu.async_copy(dst_hbm.at[wid], dst_v, sem_in)
        r0 = s * rps

        @pl.when(c == 0)
        def _():
            pltpu.sync_copy(base_hbm.at[pl.ds(r0, rps)],
                            agg_sh.at[pl.ds(r0, rps)])

        @pl.when(c != 0)
        def _():
            pltpu.sync_copy(zero_hbm.at[pl.ds(r0, rps)],
                            agg_sh.at[pl.ds(r0, rps)])

        if tail:
            @pl.when((c == 0) & (s == _NS - 1))
            def _():
                pltpu.sync_copy(base_hbm.at[pl.ds(rps * _NS, tail)],
                                agg_sh.at[pl.ds(rps * _NS, tail)])

            @pl.when((c != 0) & (s == _NS - 1))
            def _():
                pltpu.sync_copy(zero_hbm.at[pl.ds(rps * _NS, tail)],
                                agg_sh.at[pl.ds(rps * _NS, tail)])

        cp1.wait()
        cp2.wait()
        plsc.subcore_barrier()

        def block_body(q, carry):
            @pl.when(q > 0)
            def _():
                for u in range(_NBUF):
                    pltpu.make_async_copy(
                        rows_v.at[u], agg_sh.at[dst_v.at[0]], sem_s.at[u]
                    ).wait()

            for u in range(_NBUF):
                k = q * _NBUF + u
                pltpu.async_copy(m_hbm.at[src_v.at[k]], rows_v.at[u],
                                 sem_g.at[u])
                pltpu.async_copy(w16_hbm.at[wid, k], w16_v.at[u],
                                 sem_w.at[u])
            for u in range(_NBUF):
                k = q * _NBUF + u
                pltpu.make_async_copy(m_hbm.at[src_v.at[k]], rows_v.at[u],
                                      sem_g.at[u]).wait()
                pltpu.make_async_copy(w16_hbm.at[wid, k], w16_v.at[u],
                                      sem_w.at[u]).wait()

                def scale_body(i, carry2):
                    for t in range(4):
                        r = i * 4 + t
                        wv = w16_v[u, r]
                        for j in range(_HG // 16):
                            sl = pl.ds(j * 16, 16)
                            rows_v[u, r, sl] = rows_v[u, r, sl] * wv
                    return carry2

                lax.fori_loop(0, _CH // 4, scale_body, 0)
                pltpu.async_copy(rows_v.at[u], agg_sh.at[dst_v.at[k]],
                                 sem_s.at[u], add=True)
            return carry

        lax.fori_loop(0, k_chunks // _NBUF, block_body, 0)
        for u in range(_NBUF):
            pltpu.make_async_copy(
                rows_v.at[u], agg_sh.at[dst_v.at[0]], sem_s.at[u]
            ).wait()
        plsc.subcore_barrier()
        pltpu.sync_copy(agg_sh.at[pl.ds(r0, rps)],
                        out_hbm.at[c, pl.ds(r0, rps)])
        if tail:
            @pl.when(s == _NS - 1)
            def _():
                pltpu.sync_copy(agg_sh.at[pl.ds(rps * _NS, tail)],
                                out_hbm.at[c, pl.ds(rps * _NS, tail)])

    return edge_pass


def _ln_relu(h, g, b):
    mu = jnp.mean(h, axis=-1, keepdims=True)
    d = h - mu
    var = jnp.mean(d * d, axis=-1, keepdims=True)
    return jnp.maximum(d * lax.rsqrt(var + _EPS) * g + b, 0.0)


def _pre_body(x_ref, w1_ref, b1_ref, lng_ref, lnb_ref, wc_ref,
              y0_ref, y1_ref, m_ref):
    h = jnp.dot(x_ref[...], w1_ref[...],
                preferred_element_type=jnp.float32) + b1_ref[...]
    y0_ref[...] = h[:, :_HG]
    y1 = h[:, _HG:]
    y1_ref[...] = y1
    z = _ln_relu(y1, lng_ref[...], lnb_ref[...])
    m_ref[...] = jnp.dot(z, wc_ref[...], preferred_element_type=jnp.float32)


def _step_body(agg_ref, cb_ref, lng_ref, lnb_ref, wc_ref, y_ref, m_ref):
    y = agg_ref[0] + agg_ref[1] + cb_ref[...]
    y_ref[...] = y
    z = _ln_relu(y, lng_ref[...], lnb_ref[...])
    m_ref[...] = jnp.dot(z, wc_ref[...], preferred_element_type=jnp.float32)


def _last_body(agg_ref, cb_ref, y0_ref_in, fng_ref, fnb_ref, w2_ref, b2_ref,
               out_ref):
    y1 = agg_ref[0] + agg_ref[1] + cb_ref[...]
    h = jnp.concatenate([y0_ref_in[...], y1], axis=-1)
    z = _ln_relu(h, fng_ref[...], fnb_ref[...])
    out_ref[...] = jnp.dot(z, w2_ref[...],
                           preferred_element_type=jnp.float32) + b2_ref[...]


def kernel(x, edge_index_graph, edge_weight_graph, W1, b1, ln_g, ln_b,
           convW, convB, fn_g, fn_b, W2, b2):
    n = x.shape[0]
    e = edge_weight_graph.shape[0]
    out_dim = W2.shape[1]
    f32 = jnp.float32

    k_chunks = -(-e // (_NW * _CH))
    k_chunks = -(-k_chunks // _NBUF) * _NBUF
    ep = _NW * k_chunks * _CH
    spread = (jnp.arange(ep - e, dtype=jnp.int32) * 64) % n
    src = jnp.concatenate([edge_index_graph[0], spread]).reshape(
        _NW, k_chunks, _CH)
    dst = jnp.concatenate([edge_index_graph[1], spread]).reshape(
        _NW, k_chunks, _CH)
    wgt = jnp.pad(edge_weight_graph, (0, ep - e))
    w16 = jnp.broadcast_to(wgt[:, None], (ep, 16)).reshape(
        _NW, k_chunks, _CH, 16)
    zeros = jnp.zeros((n, _HG), f32)

    edge_pass = _make_edge_pass(n, k_chunks)
    sds = jax.ShapeDtypeStruct

    y0, y1, m = pl.pallas_call(
        _pre_body,
        out_shape=(sds((n, _HG), f32), sds((n, _HG), f32), sds((n, _HG), f32)),
    )(x, W1, b1[None], ln_g[0, 0][None], ln_b[0, 0][None], convW[0, 0])

    steps = [(l, g) for l in range(_L) for g in range(_G)]
    for idx, (l, g) in enumerate(steps):
        base = y0 if g == 0 else y1
        agg2 = edge_pass(m, src, dst, w16, base, zeros)
        if idx + 1 < len(steps):
            ln_, gn_ = steps[idx + 1]
            y, m = pl.pallas_call(
                _step_body,
                out_shape=(sds((n, _HG), f32), sds((n, _HG), f32)),
            )(agg2, convB[l, g][None], ln_g[ln_, gn_][None],
              ln_b[ln_, gn_][None], convW[ln_, gn_])
            if g == 0:
                y0 = y
            else:
                y1 = y
        else:
            out = pl.pallas_call(
                _last_body,
                out_shape=sds((n, out_dim), f32),
            )(agg2, convB[l, g][None], y0, fn_g[None], fn_b[None], W2, b2[None])
    return out


# speedup vs baseline: 1.1483x; 1.0144x over previous
import functools

import jax
import jax.numpy as jnp
from jax import lax
from jax.experimental import pallas as pl
from jax.experimental.pallas import tpu as pltpu
from jax.experimental.pallas import tpu_sc as plsc

_HG = 64
_L = 4
_G = 2
_NC = 2
_NS = 16
_NW = _NC * _NS
_CH = 128
_NBUF = 6
_EPS = 1e-5


def _make_edge_pass(n, k_chunks):
    mesh = plsc.VectorSubcoreMesh(core_axis_name="c", subcore_axis_name="s")
    rps = (n // _NS) // 8 * 8
    tail = n - rps * _NS

    @functools.partial(
        pl.kernel,
        out_type=jax.ShapeDtypeStruct((_NC, n, _HG), jnp.float32),
        mesh=mesh,
        scratch_types=[
            pltpu.VMEM((k_chunks, _CH), jnp.int32),
            pltpu.VMEM((k_chunks, _CH), jnp.int32),
            pltpu.VMEM((_NBUF, _CH, 16), jnp.float32),
            pltpu.VMEM((_NBUF, _CH, _HG), jnp.float32),
            pltpu.VMEM_SHARED((n, _HG), jnp.float32),
            pltpu.SemaphoreType.DMA,
            pltpu.SemaphoreType.DMA((_NBUF,)),
            pltpu.SemaphoreType.DMA((_NBUF,)),
            pltpu.SemaphoreType.DMA((_NBUF,)),
        ],
        compiler_params=pltpu.CompilerParams(use_tc_tiling_on_sc=False),
    )
    def edge_pass(m_hbm, src_hbm, dst_hbm, w16_hbm, base_hbm, zero_hbm,
                  out_hbm, src_v, dst_v, w16_v, rows_v, agg_sh,
                  sem_in, sem_g, sem_s, sem_w):
        c = lax.axis_index("c")
        s = lax.axis_index("s")
        wid = c * _NS + s
        cp1 = pltpu.async_copy(src_hbm.at[wid], src_v, sem_in)
        cp2 = pltpu.async_copy(dst_hbm.at[wid], dst_v, sem_in)
        r0 = s * rps

        @pl.when(c == 0)
        def _():
            pltpu.sync_copy(base_hbm.at[pl.ds(r0, rps)],
                            agg_sh.at[pl.ds(r0, rps)])

        @pl.when(c != 0)
        def _():
            pltpu.sync_copy(zero_hbm.at[pl.ds(r0, rps)],
                            agg_sh.at[pl.ds(r0, rps)])

        if tail:
            @pl.when((c == 0) & (s == _NS - 1))
            def _():
                pltpu.sync_copy(base_hbm.at[pl.ds(rps * _NS, tail)],
                                agg_sh.at[pl.ds(rps * _NS, tail)])

            @pl.when((c != 0) & (s == _NS - 1))
            def _():
                pltpu.sync_copy(zero_hbm.at[pl.ds(rps * _NS, tail)],
                                agg_sh.at[pl.ds(rps * _NS, tail)])

        cp1.wait()
        cp2.wait()
        plsc.subcore_barrier()

        def block_body(q, carry):
            @pl.when(q > 0)
            def _():
                for u in range(_NBUF):
                    pltpu.make_async_copy(
                        rows_v.at[u], agg_sh.at[dst_v.at[0]], sem_s.at[u]
                    ).wait()

            for u in range(_NBUF):
                k = q * _NBUF + u
                pltpu.async_copy(m_hbm.at[src_v.at[k]], rows_v.at[u],
                                 sem_g.at[u])
                pltpu.async_copy(w16_hbm.at[wid, k], w16_v.at[u],
                                 sem_w.at[u])
            for u in range(_NBUF):
                k = q * _NBUF + u
                pltpu.make_async_copy(m_hbm.at[src_v.at[k]], rows_v.at[u],
                                      sem_g.at[u]).wait()
                pltpu.make_async_copy(w16_hbm.at[wid, k], w16_v.at[u],
                                      sem_w.at[u]).wait()

                def scale_body(i, carry2):
                    for t in range(2):
                        r = i * 2 + t
                        wv = w16_v[u, r]
                        for j in range(_HG // 16):
                            sl = pl.ds(j * 16, 16)
                            rows_v[u, r, sl] = rows_v[u, r, sl] * wv
                    return carry2

                lax.fori_loop(0, _CH // 2, scale_body, 0)
                pltpu.async_copy(rows_v.at[u], agg_sh.at[dst_v.at[k]],
                                 sem_s.at[u], add=True)
            return carry

        lax.fori_loop(0, k_chunks // _NBUF, block_body, 0)
        for u in range(_NBUF):
            pltpu.make_async_copy(
                rows_v.at[u], agg_sh.at[dst_v.at[0]], sem_s.at[u]
            ).wait()
        plsc.subcore_barrier()
        pltpu.sync_copy(agg_sh.at[pl.ds(r0, rps)],
                        out_hbm.at[c, pl.ds(r0, rps)])
        if tail:
            @pl.when(s == _NS - 1)
            def _():
                pltpu.sync_copy(agg_sh.at[pl.ds(rps * _NS, tail)],
                                out_hbm.at[c, pl.ds(rps * _NS, tail)])

    return edge_pass


def _ln_relu(h, g, b):
    mu = jnp.mean(h, axis=-1, keepdims=True)
    d = h - mu
    var = jnp.mean(d * d, axis=-1, keepdims=True)
    return jnp.maximum(d * lax.rsqrt(var + _EPS) * g + b, 0.0)


def _pre_body(x_ref, w1_ref, b1_ref, lng_ref, lnb_ref, wc_ref,
              y0_ref, y1_ref, m_ref):
    h = jnp.dot(x_ref[...], w1_ref[...],
                preferred_element_type=jnp.float32) + b1_ref[...]
    y0_ref[...] = h[:, :_HG]
    y1 = h[:, _HG:]
    y1_ref[...] = y1
    z = _ln_relu(y1, lng_ref[...], lnb_ref[...])
    m_ref[...] = jnp.dot(z, wc_ref[...], preferred_element_type=jnp.float32)


def _step_body(agg_ref, cb_ref, lng_ref, lnb_ref, wc_ref, y_ref, m_ref):
    y = agg_ref[0] + agg_ref[1] + cb_ref[...]
    y_ref[...] = y
    z = _ln_relu(y, lng_ref[...], lnb_ref[...])
    m_ref[...] = jnp.dot(z, wc_ref[...], preferred_element_type=jnp.float32)


def _last_body(agg_ref, cb_ref, y0_ref_in, fng_ref, fnb_ref, w2_ref, b2_ref,
               out_ref):
    y1 = agg_ref[0] + agg_ref[1] + cb_ref[...]
    h = jnp.concatenate([y0_ref_in[...], y1], axis=-1)
    z = _ln_relu(h, fng_ref[...], fnb_ref[...])
    out_ref[...] = jnp.dot(z, w2_ref[...],
                           preferred_element_type=jnp.float32) + b2_ref[...]


def kernel(x, edge_index_graph, edge_weight_graph, W1, b1, ln_g, ln_b,
           convW, convB, fn_g, fn_b, W2, b2):
    n = x.shape[0]
    e = edge_weight_graph.shape[0]
    out_dim = W2.shape[1]
    f32 = jnp.float32

    k_chunks = -(-e // (_NW * _CH))
    k_chunks = -(-k_chunks // _NBUF) * _NBUF
    ep = _NW * k_chunks * _CH
    spread = (jnp.arange(ep - e, dtype=jnp.int32) * 64) % n
    src = jnp.concatenate([edge_index_graph[0], spread]).reshape(
        _NW, k_chunks, _CH)
    dst = jnp.concatenate([edge_index_graph[1], spread]).reshape(
        _NW, k_chunks, _CH)
    wgt = jnp.pad(edge_weight_graph, (0, ep - e))
    w16 = jnp.broadcast_to(wgt[:, None], (ep, 16)).reshape(
        _NW, k_chunks, _CH, 16)
    zeros = jnp.zeros((n, _HG), f32)

    edge_pass = _make_edge_pass(n, k_chunks)
    sds = jax.ShapeDtypeStruct

    y0, y1, m = pl.pallas_call(
        _pre_body,
        out_shape=(sds((n, _HG), f32), sds((n, _HG), f32), sds((n, _HG), f32)),
    )(x, W1, b1[None], ln_g[0, 0][None], ln_b[0, 0][None], convW[0, 0])

    steps = [(l, g) for l in range(_L) for g in range(_G)]
    for idx, (l, g) in enumerate(steps):
        base = y0 if g == 0 else y1
        agg2 = edge_pass(m, src, dst, w16, base, zeros)
        if idx + 1 < len(steps):
            ln_, gn_ = steps[idx + 1]
            y, m = pl.pallas_call(
                _step_body,
                out_shape=(sds((n, _HG), f32), sds((n, _HG), f32)),
            )(agg2, convB[l, g][None], ln_g[ln_, gn_][None],
              ln_b[ln_, gn_][None], convW[ln_, gn_])
            if g == 0:
                y0 = y
            else:
                y1 = y
        else:
            out = pl.pallas_call(
                _last_body,
                out_shape=sds((n, out_dim), f32),
            )(agg2, convB[l, g][None], y0, fn_g[None], fn_b[None], W2, b2[None])
    return out
